# baseline (speedup 1.0000x reference)
import functools

import jax
import jax.numpy as jnp
from jax import lax
from jax.experimental import pallas as pl
from jax.experimental.pallas import tpu as pltpu
from jax.experimental.pallas import tpu_sc as plsc

N_NODES = 10000
D = 128
H = 64
E = 320000

NS = 16
BLK = 128
NBLK = 160
NDEEP = 4
NBLK_IDX = NBLK + NDEEP
EP_TILE = NBLK_IDX * BLK
PAD_N = 10240
ROWS_PER_TILE = PAD_N // NS
CHUNK = 128
NCHUNK = ROWS_PER_TILE // CHUNK

BM = 1000


def _mm_body(xu_ref, xr_ref, wu_ref, wr_ref, bu_ref, br_ref,
             y_ref, z_ref):
    tu = jnp.dot(xu_ref[...], wu_ref[...],
                 preferred_element_type=jnp.float32) + bu_ref[...]
    tr = jnp.dot(xr_ref[...], wr_ref[...],
                 preferred_element_type=jnp.float32) + br_ref[...]
    y_ref[0] = tu[:, :H]
    y_ref[1] = tr[:, :H]
    z_ref[0] = tr[:, H:]
    z_ref[1] = tu[:, H:]


def _dense_project(xu, xr, wu, wr, bu, br):
    return pl.pallas_call(
        _mm_body,
        grid=(N_NODES // BM,),
        in_specs=[
            pl.BlockSpec((BM, D), lambda i: (i, 0)),
            pl.BlockSpec((BM, D), lambda i: (i, 0)),
            pl.BlockSpec((D, 2 * H), lambda i: (0, 0)),
            pl.BlockSpec((D, 2 * H), lambda i: (0, 0)),
            pl.BlockSpec((1, 2 * H), lambda i: (0, 0)),
            pl.BlockSpec((1, 2 * H), lambda i: (0, 0)),
        ],
        out_specs=[
            pl.BlockSpec((2, BM, H), lambda i: (0, i, 0)),
            pl.BlockSpec((2, BM, H), lambda i: (0, i, 0)),
        ],
        out_shape=[
            jax.ShapeDtypeStruct((2, N_NODES, H), jnp.float32),
            jax.ShapeDtypeStruct((2, PAD_N, H), jnp.float32),
        ],
    )(xu, xr, wu, wr, bu, br)


_sc_mesh = plsc.VectorSubcoreMesh(core_axis_name="c", subcore_axis_name="s")


@functools.partial(
    pl.kernel,
    out_type=jax.ShapeDtypeStruct((2, PAD_N, H), jnp.float32),
    mesh=_sc_mesh,
    scratch_types=[
        pltpu.VMEM((NBLK_IDX, BLK), jnp.int32),
        pltpu.VMEM((NBLK_IDX, BLK), jnp.int32),
        pltpu.VMEM((BLK, H), jnp.float32),
        pltpu.VMEM((BLK, H), jnp.float32),
        pltpu.VMEM((BLK, H), jnp.float32),
        pltpu.VMEM((BLK, H), jnp.float32),
        pltpu.VMEM((CHUNK, H), jnp.float32),
        pltpu.VMEM_SHARED((PAD_N, H), jnp.float32),
        pltpu.SemaphoreType.DMA,
        pltpu.SemaphoreType.DMA,
        pltpu.SemaphoreType.DMA,
        pltpu.SemaphoreType.DMA,
    ],
    compiler_params=pltpu.CompilerParams(use_tc_tiling_on_sc=False),
)
def _sc_scatter(y_tab, s_all, d_all, acc_out,
                src_v, dst_v, rows_v0, rows_v1, rows_v2, rows_v3,
                zblk64, acc_sh, sem0, sem1, sem2, sem3):
    c = lax.axis_index("c")
    s = lax.axis_index("s")

    zeros16 = jnp.zeros((16,), jnp.float32)

    def fill_row(i, carry):
        for k in range(H // 16):
            zblk64[i, pl.ds(k * 16, 16)] = zeros16
        return carry

    lax.fori_loop(0, CHUNK, fill_row, 0)

    base = s * ROWS_PER_TILE
    for t in range(NCHUNK):
        pltpu.sync_copy(zblk64, acc_sh.at[pl.ds(base + t * CHUNK, CHUNK)])

    pltpu.sync_copy(s_all.at[c, s], src_v)
    pltpu.sync_copy(d_all.at[c, s], dst_v)
    plsc.subcore_barrier()

    bufs = (rows_v0, rows_v1, rows_v2, rows_v3)
    sems = (sem0, sem1, sem2, sem3)
    for b in range(NDEEP):
        pltpu.async_copy(y_tab.at[src_v.at[b]], bufs[b], sems[b])

    def edge_quad(i, carry):
        j = NDEEP * i
        for b in range(NDEEP):
            pltpu.make_async_copy(y_tab.at[src_v.at[j + b]], bufs[b],
                                  sems[b]).wait()
            pltpu.sync_copy(bufs[b], acc_sh.at[dst_v.at[j + b]], add=True)
            pltpu.async_copy(y_tab.at[src_v.at[j + b + NDEEP]], bufs[b],
                             sems[b])
        return carry

    lax.fori_loop(0, NBLK // NDEEP, edge_quad, 0)
    for b in range(NDEEP):
        pltpu.make_async_copy(y_tab.at[src_v.at[NBLK + b]], bufs[b],
                              sems[b]).wait()
    plsc.subcore_barrier()

    for t in range(NCHUNK):
        r0 = base + t * CHUNK
        pltpu.sync_copy(acc_sh.at[pl.ds(r0, CHUNK)], zblk64)
        pltpu.sync_copy(zblk64, acc_out.at[c].at[pl.ds(r0, CHUNK)])


@functools.partial(
    pl.kernel,
    out_type=jax.ShapeDtypeStruct((2, PAD_N, H), jnp.float32),
    mesh=_sc_mesh,
    scratch_types=[
        pltpu.VMEM((NBLK_IDX, BLK), jnp.int32),
        pltpu.VMEM((BLK, 16), jnp.float32),
        pltpu.VMEM((CHUNK, 16), jnp.float32),
        pltpu.VMEM((CHUNK, H), jnp.float32),
        pltpu.VMEM((CHUNK, 16), jnp.float32),
        pltpu.VMEM((CHUNK, H), jnp.float32),
        pltpu.VMEM((CHUNK, H), jnp.float32),
        pltpu.VMEM_SHARED((PAD_N, 16), jnp.float32),
    ],
    compiler_params=pltpu.CompilerParams(use_tc_tiling_on_sc=False),
)
def _sc_finalize(acc_all, z_all, d_all, out,
                 dst_v, ones_v, zblk16, accv, cntv, zv, outv, cnt_sh):
    c = lax.axis_index("c")
    s = lax.axis_index("s")

    zeros16 = jnp.zeros((16,), jnp.float32)
    ones16 = jnp.ones((16,), jnp.float32)

    def fill_row(i, carry):
        zblk16[i, :] = zeros16
        ones_v[i, :] = ones16
        return carry

    lax.fori_loop(0, CHUNK, fill_row, 0)

    base = s * ROWS_PER_TILE
    for t in range(NCHUNK):
        pltpu.sync_copy(zblk16, cnt_sh.at[pl.ds(base + t * CHUNK, CHUNK)])

    pltpu.sync_copy(d_all.at[c, s], dst_v)
    plsc.subcore_barrier()

    def edge_block(j, carry):
        pltpu.sync_copy(ones_v, cnt_sh.at[dst_v.at[j]], add=True)
        return carry

    lax.fori_loop(0, NBLK, edge_block, 0)
    plsc.subcore_barrier()

    def chunk(t, carry):
        r0 = base + t * CHUNK
        pltpu.sync_copy(acc_all.at[c].at[pl.ds(r0, CHUNK)], accv)
        pltpu.sync_copy(cnt_sh.at[pl.ds(r0, CHUNK)], cntv)
        pltpu.sync_copy(z_all.at[c].at[pl.ds(r0, CHUNK)], zv)

        def row(i, carry2):
            inv = 1.0 / jnp.maximum(cntv[i, :], 1.0)
            for k in range(H // 16):
                sl = pl.ds(k * 16, 16)
                v = accv[i, sl] * inv + zv[i, sl]
                outv[i, sl] = jnp.maximum(v, 0.0)
            return carry2

        lax.fori_loop(0, CHUNK, row, 0)
        pltpu.sync_copy(outv, out.at[c].at[pl.ds(r0, CHUNK)])
        return carry

    lax.fori_loop(0, NCHUNK, chunk, 0)


def _edge_splits(ei, src_off):
    ei = ei.astype(jnp.int32)
    src = ei[0].reshape(NS, E // NS) + src_off
    dst = ei[1].reshape(NS, E // NS)
    pad = EP_TILE - E // NS
    src = jnp.pad(src, ((0, 0), (0, pad)),
                  constant_values=src_off)
    dst = jnp.pad(dst, ((0, 0), (0, pad)),
                  constant_values=N_NODES)
    return (src.reshape(NS, NBLK_IDX, BLK),
            dst.reshape(NS, NBLK_IDX, BLK))


def kernel(x_user, x_resource, edge_index_user_accessed_resource,
           edge_index_resource_rev_accessed_user,
           Wl_ur, Wr_ur, b_ur, Wl_ru, Wr_ru, b_ru):
    wu = jnp.concatenate([Wl_ur, Wr_ru], axis=1)
    wr = jnp.concatenate([Wl_ru, Wr_ur], axis=1)
    zeros_h = jnp.zeros((H,), jnp.float32)
    bu = jnp.concatenate([zeros_h, b_ru])[None, :]
    br = jnp.concatenate([zeros_h, b_ur])[None, :]

    y_tab, z_all = _dense_project(x_user, x_resource, wu, wr, bu, br)
    y_flat = y_tab.reshape(2 * N_NODES, H)

    sa, da = _edge_splits(edge_index_user_accessed_resource, 0)
    sb, db = _edge_splits(edge_index_resource_rev_accessed_user, N_NODES)
    s_all = jnp.stack([sa, sb])
    d_all = jnp.stack([da, db])

    acc_all = _sc_scatter(y_flat, s_all, d_all)
    out = _sc_finalize(acc_all, z_all, d_all)
    return (out[1, :N_NODES], out[0, :N_NODES])


# speedup vs baseline: 1.1775x; 1.1775x over previous
import functools

import jax
import jax.numpy as jnp
from jax import lax
from jax.experimental import pallas as pl
from jax.experimental.pallas import tpu as pltpu
from jax.experimental.pallas import tpu_sc as plsc

N_NODES = 10000
D = 128
H = 64
E = 320000

NS = 16
BLK = 128
SUPER = 1
SBLK = SUPER * BLK
NSUP = 158
NSUP_IDX = NSUP + 2
NBLK = NSUP * SUPER
EP_TILE = NSUP_IDX * SBLK
PAD_N = 10240
ROWS_PER_TILE = PAD_N // NS
CHUNK = 128
NCHUNK = ROWS_PER_TILE // CHUNK

BM = 1000


def _mm_body(xu_ref, xr_ref, wu_ref, wr_ref, bu_ref, br_ref,
             y_ref, z_ref):
    tu = jnp.dot(xu_ref[...], wu_ref[...],
                 preferred_element_type=jnp.float32) + bu_ref[...]
    tr = jnp.dot(xr_ref[...], wr_ref[...],
                 preferred_element_type=jnp.float32) + br_ref[...]
    y_ref[0] = tu[:, :H]
    y_ref[1] = tr[:, :H]
    z_ref[0] = tr[:, H:]
    z_ref[1] = tu[:, H:]


def _dense_project(xu, xr, wu, wr, bu, br):
    return pl.pallas_call(
        _mm_body,
        grid=(N_NODES // BM,),
        in_specs=[
            pl.BlockSpec((BM, D), lambda i: (i, 0)),
            pl.BlockSpec((BM, D), lambda i: (i, 0)),
            pl.BlockSpec((D, 2 * H), lambda i: (0, 0)),
            pl.BlockSpec((D, 2 * H), lambda i: (0, 0)),
            pl.BlockSpec((1, 2 * H), lambda i: (0, 0)),
            pl.BlockSpec((1, 2 * H), lambda i: (0, 0)),
        ],
        out_specs=[
            pl.BlockSpec((2, BM, H), lambda i: (0, i, 0)),
            pl.BlockSpec((2, BM, H), lambda i: (0, i, 0)),
        ],
        out_shape=[
            jax.ShapeDtypeStruct((2, N_NODES, H), jnp.float32),
            jax.ShapeDtypeStruct((2, PAD_N, H), jnp.float32),
        ],
    )(xu, xr, wu, wr, bu, br)


def _split_body(o_ref, z_ref, u_ref, r_ref):
    u_ref[...] = jnp.maximum(o_ref[1] + z_ref[1], 0.0)
    r_ref[...] = jnp.maximum(o_ref[0] + z_ref[0], 0.0)


def _split_outputs(out_sc, z_all):
    return pl.pallas_call(
        _split_body,
        grid=(N_NODES // BM,),
        in_specs=[pl.BlockSpec((2, BM, H), lambda i: (0, i, 0)),
                  pl.BlockSpec((2, BM, H), lambda i: (0, i, 0))],
        out_specs=[
            pl.BlockSpec((BM, H), lambda i: (i, 0)),
            pl.BlockSpec((BM, H), lambda i: (i, 0)),
        ],
        out_shape=[
            jax.ShapeDtypeStruct((N_NODES, H), jnp.float32),
            jax.ShapeDtypeStruct((N_NODES, H), jnp.float32),
        ],
    )(out_sc, z_all)


_sc_mesh = plsc.VectorSubcoreMesh(core_axis_name="c", subcore_axis_name="s")


@functools.partial(
    pl.kernel,
    out_type=pltpu.HBM((2, PAD_N, H), jnp.float32),
    mesh=_sc_mesh,
    scratch_types=[
        pltpu.VMEM((NSUP_IDX, SBLK), jnp.int32),
        pltpu.VMEM((NSUP_IDX * SUPER, BLK), jnp.int32),
        pltpu.VMEM((SBLK, H), jnp.float32),
        pltpu.VMEM((SBLK, H), jnp.float32),
        pltpu.VMEM((BLK, 16), jnp.float32),
        pltpu.VMEM((CHUNK, H), jnp.float32),
        pltpu.VMEM((CHUNK, 16), jnp.float32),
        pltpu.VMEM((CHUNK, H), jnp.float32),
        pltpu.VMEM((CHUNK, 16), jnp.float32),
        pltpu.VMEM((CHUNK, H), jnp.float32),
        pltpu.VMEM_SHARED((PAD_N, H), jnp.float32),
        pltpu.VMEM_SHARED((PAD_N, 16), jnp.float32),
        pltpu.SemaphoreType.DMA,
        pltpu.SemaphoreType.DMA,
    ],
    compiler_params=pltpu.CompilerParams(use_tc_tiling_on_sc=False),
)
def _sc_aggregate(y_tab, s_all, d_all, out,
                  src_v, dst_v, rows_v0, rows_v1, ones_v, zblk64, zblk16,
                  accv, cntv, outv, acc_sh, cnt_sh, sem0, sem1):
    c = lax.axis_index("c")
    s = lax.axis_index("s")

    zeros16 = jnp.zeros((16,), jnp.float32)
    ones16 = jnp.ones((16,), jnp.float32)

    def fill_row(i, carry):
        for k in range(H // 16):
            zblk64[i, pl.ds(k * 16, 16)] = zeros16
        zblk16[i, :] = zeros16
        ones_v[i, :] = ones16
        return carry

    lax.fori_loop(0, CHUNK, fill_row, 0)

    base = s * ROWS_PER_TILE
    for t in range(NCHUNK):
        pltpu.sync_copy(zblk64, acc_sh.at[pl.ds(base + t * CHUNK, CHUNK)])
        pltpu.sync_copy(zblk16, cnt_sh.at[pl.ds(base + t * CHUNK, CHUNK)])

    def stage_idx(t, carry):
        sl = pl.ds(t * 16, 16)
        pltpu.sync_copy(s_all.at[c, s].at[sl], src_v.at[sl])
        pltpu.sync_copy(d_all.at[c, s].at[sl], dst_v.at[sl])
        return carry

    lax.fori_loop(0, NSUP_IDX // 16, stage_idx, 0)
    plsc.subcore_barrier()

    def scat(buf, g):
        for b in range(SUPER):
            blk = g * SUPER + b
            pltpu.sync_copy(buf.at[pl.ds(b * BLK, BLK)],
                            acc_sh.at[dst_v.at[blk]], add=True)
            pltpu.sync_copy(ones_v, cnt_sh.at[dst_v.at[blk]], add=True)

    pltpu.async_copy(y_tab.at[src_v.at[0]], rows_v0, sem0)

    def edge_super(i, carry):
        pltpu.make_async_copy(y_tab.at[src_v.at[i]], rows_v0, sem0).wait()
        scat(rows_v0, i)
        pltpu.async_copy(y_tab.at[src_v.at[i + 1]], rows_v0, sem0)
        return carry

    lax.fori_loop(0, NSUP, edge_super, 0)
    pltpu.make_async_copy(y_tab.at[src_v.at[NSUP]], rows_v0, sem0).wait()
    plsc.subcore_barrier()

    def chunk(t, carry):
        r0 = base + t * CHUNK
        pltpu.sync_copy(acc_sh.at[pl.ds(r0, CHUNK)], accv)
        pltpu.sync_copy(cnt_sh.at[pl.ds(r0, CHUNK)], cntv)

        def row(i, carry2):
            inv = 1.0 / jnp.maximum(cntv[i, :], 1.0)
            for k in range(H // 16):
                sl = pl.ds(k * 16, 16)
                outv[i, sl] = accv[i, sl] * inv
            return carry2

        lax.fori_loop(0, CHUNK, row, 0)
        pltpu.sync_copy(outv, out.at[c].at[pl.ds(r0, CHUNK)])
        return carry

    lax.fori_loop(0, NCHUNK, chunk, 0)


def _edge_splits(ei, src_off):
    ei = ei.astype(jnp.int32)
    src = ei[0].reshape(NS, E // NS) + src_off
    dst = ei[1].reshape(NS, E // NS)
    pad = EP_TILE - E // NS
    src = jnp.pad(src, ((0, 0), (0, pad)),
                  constant_values=src_off)
    dst = jnp.pad(dst, ((0, 0), (0, pad)),
                  constant_values=N_NODES)
    return (src.reshape(NS, NSUP_IDX, SBLK),
            dst.reshape(NS, NSUP_IDX * SUPER, BLK))


def kernel(x_user, x_resource, edge_index_user_accessed_resource,
           edge_index_resource_rev_accessed_user,
           Wl_ur, Wr_ur, b_ur, Wl_ru, Wr_ru, b_ru):
    wu = jnp.concatenate([Wl_ur, Wr_ru], axis=1)
    wr = jnp.concatenate([Wl_ru, Wr_ur], axis=1)
    zeros_h = jnp.zeros((H,), jnp.float32)
    bu = jnp.concatenate([zeros_h, b_ru])[None, :]
    br = jnp.concatenate([zeros_h, b_ur])[None, :]

    y_tab, z_all = _dense_project(x_user, x_resource, wu, wr, bu, br)
    y_flat = y_tab.reshape(2 * N_NODES, H)

    sa, da = _edge_splits(edge_index_user_accessed_resource, 0)
    sb, db = _edge_splits(edge_index_resource_rev_accessed_user, N_NODES)
    s_all = jnp.stack([sa, sb])
    d_all = jnp.stack([da, db])

    out_sc = _sc_aggregate(y_flat, s_all, d_all)
    out_user, out_res = _split_outputs(out_sc, z_all)
    return (out_user, out_res)


# speedup vs baseline: 1.4403x; 1.2233x over previous
import functools

import jax
import jax.numpy as jnp
from jax import lax
from jax.experimental import pallas as pl
from jax.experimental.pallas import tpu as pltpu
from jax.experimental.pallas import tpu_sc as plsc

N_NODES = 10000
D = 128
H = 64
E = 320000

NS = 16
BLK = 128
NBLK = 158
NBLK_IDX = NBLK + 2
EP_TILE = NBLK_IDX * BLK
PAD_N = 10240
ROWS_PER_TILE = PAD_N // NS
CHUNK = 128
NCHUNK = ROWS_PER_TILE // CHUNK

BM = 1000


def _mm_body(xu_ref, xr_ref, wu_ref, wr_ref, bu_ref, br_ref,
             y_ref, z_ref):
    tu = jnp.dot(xu_ref[...], wu_ref[...],
                 preferred_element_type=jnp.float32) + bu_ref[...]
    tr = jnp.dot(xr_ref[...], wr_ref[...],
                 preferred_element_type=jnp.float32) + br_ref[...]
    y_ref[0] = tu[:, :H]
    y_ref[1] = tr[:, :H]
    z_ref[0] = tr[:, H:]
    z_ref[1] = tu[:, H:]


def _dense_project(xu, xr, wu, wr, bu, br):
    return pl.pallas_call(
        _mm_body,
        grid=(N_NODES // BM,),
        in_specs=[
            pl.BlockSpec((BM, D), lambda i: (i, 0)),
            pl.BlockSpec((BM, D), lambda i: (i, 0)),
            pl.BlockSpec((D, 2 * H), lambda i: (0, 0)),
            pl.BlockSpec((D, 2 * H), lambda i: (0, 0)),
            pl.BlockSpec((1, 2 * H), lambda i: (0, 0)),
            pl.BlockSpec((1, 2 * H), lambda i: (0, 0)),
        ],
        out_specs=[
            pl.BlockSpec((2, BM, H), lambda i: (0, i, 0)),
            pl.BlockSpec((2, BM, H), lambda i: (0, i, 0)),
        ],
        out_shape=[
            jax.ShapeDtypeStruct((2, N_NODES, H), jnp.float32),
            jax.ShapeDtypeStruct((2, N_NODES, H), jnp.float32),
        ],
    )(xu, xr, wu, wr, bu, br)


def _split_body(o_ref, z_ref, u_ref, r_ref):
    u_ref[...] = jnp.maximum(o_ref[1] + z_ref[1], 0.0)
    r_ref[...] = jnp.maximum(o_ref[0] + z_ref[0], 0.0)


def _split_outputs(out_sc, z_all):
    return pl.pallas_call(
        _split_body,
        grid=(N_NODES // BM,),
        in_specs=[pl.BlockSpec((2, BM, H), lambda i: (0, i, 0)),
                  pl.BlockSpec((2, BM, H), lambda i: (0, i, 0))],
        out_specs=[
            pl.BlockSpec((BM, H), lambda i: (i, 0)),
            pl.BlockSpec((BM, H), lambda i: (i, 0)),
        ],
        out_shape=[
            jax.ShapeDtypeStruct((N_NODES, H), jnp.float32),
            jax.ShapeDtypeStruct((N_NODES, H), jnp.float32),
        ],
    )(out_sc, z_all)


_sc_mesh = plsc.VectorSubcoreMesh(core_axis_name="c", subcore_axis_name="s")


@functools.partial(
    pl.kernel,
    out_type=(pltpu.HBM((2, PAD_N, H), jnp.float32),
              pltpu.HBM((2, NS, PAD_N), jnp.float32)),
    mesh=_sc_mesh,
    scratch_types=[
        pltpu.VMEM((NBLK_IDX, BLK), jnp.int32),
        pltpu.VMEM((4, BLK), jnp.int32),
        pltpu.VMEM((2, BLK), jnp.int32),
        pltpu.VMEM((BLK, H), jnp.float32),
        pltpu.VMEM((BLK, H), jnp.float32),
        pltpu.VMEM((CHUNK, H), jnp.float32),
        pltpu.VMEM((CHUNK, H), jnp.float32),
        pltpu.VMEM((CHUNK, H), jnp.float32),
        pltpu.VMEM((PAD_N,), jnp.float32),
        pltpu.VMEM((NS, CHUNK), jnp.float32),
        pltpu.VMEM((CHUNK,), jnp.float32),
        pltpu.VMEM_SHARED((PAD_N, H), jnp.float32),
        pltpu.SemaphoreType.DMA,
        pltpu.SemaphoreType.DMA,
    ],
    compiler_params=pltpu.CompilerParams(use_tc_tiling_on_sc=False,
                                         needs_layout_passes=False),
)
def _sc_aggregate(y_tab, p_all, out, cnt_out,
                  pk_v, srcb, dstb, rows_v0, rows_v1, zblk64,
                  accv, outv, lcnt, cmrg, csum, acc_sh, sem0, sem1):
    c = lax.axis_index("c")
    s = lax.axis_index("s")

    zeros16 = jnp.zeros((16,), jnp.float32)
    ones16 = jnp.ones((16,), jnp.float32)

    def fill_row(i, carry):
        for k in range(H // 16):
            zblk64[i, pl.ds(k * 16, 16)] = zeros16
        return carry

    lax.fori_loop(0, CHUNK, fill_row, 0)

    def zero_cnt(i, carry):
        lcnt[pl.ds(i * 16, 16)] = zeros16
        return carry

    lax.fori_loop(0, PAD_N // 16, zero_cnt, 0)

    base = s * ROWS_PER_TILE
    for t in range(NCHUNK):
        pltpu.sync_copy(zblk64, acc_sh.at[pl.ds(base + t * CHUNK, CHUNK)])

    pltpu.sync_copy(p_all.at[c, s], pk_v)
    plsc.subcore_barrier()

    def unpack_src(j, slot):
        for k in range(BLK // 16):
            sl = pl.ds(k * 16, 16)
            srcb[slot, sl] = jnp.bitwise_and(pk_v[j, sl], 32767)

    def unpack_dst_and_count(j, slot):
        for k in range(BLK // 16):
            sl = pl.ds(k * 16, 16)
            dvec = jax.lax.shift_right_logical(pk_v[j, sl], 15)
            dstb[slot, sl] = dvec
            plsc.addupdate_scatter(lcnt, [dvec], ones16)

    unpack_src(0, 0)
    pltpu.async_copy(y_tab.at[srcb.at[0]], rows_v0, sem0)
    unpack_src(1, 1)
    pltpu.async_copy(y_tab.at[srcb.at[1]], rows_v1, sem1)

    def edge_pair(i, carry):
        j = 2 * i
        pltpu.make_async_copy(y_tab.at[srcb.at[j & 3]], rows_v0,
                              sem0).wait()
        unpack_dst_and_count(j, 0)
        pltpu.sync_copy(rows_v0, acc_sh.at[dstb.at[0]], add=True)
        unpack_src(j + 2, (j + 2) & 3)
        pltpu.async_copy(y_tab.at[srcb.at[(j + 2) & 3]], rows_v0, sem0)
        pltpu.make_async_copy(y_tab.at[srcb.at[(j + 1) & 3]], rows_v1,
                              sem1).wait()
        unpack_dst_and_count(j + 1, 1)
        pltpu.sync_copy(rows_v1, acc_sh.at[dstb.at[1]], add=True)
        unpack_src(j + 3, (j + 3) & 3)
        pltpu.async_copy(y_tab.at[srcb.at[(j + 3) & 3]], rows_v1, sem1)
        return carry

    lax.fori_loop(0, NBLK // 2, edge_pair, 0)
    pltpu.make_async_copy(y_tab.at[srcb.at[NBLK & 3]], rows_v0,
                          sem0).wait()
    pltpu.make_async_copy(y_tab.at[srcb.at[(NBLK + 1) & 3]], rows_v1,
                          sem1).wait()
    pltpu.sync_copy(lcnt, cnt_out.at[c, s])
    plsc.subcore_barrier()

    def chunk(t, carry):
        r0 = base + t * CHUNK
        pltpu.sync_copy(acc_sh.at[pl.ds(r0, CHUNK)], accv)
        pltpu.sync_copy(cnt_out.at[c, :, pl.ds(r0, CHUNK)], cmrg)

        def colsum(k, carry2):
            sl = pl.ds(k * 16, 16)

            def accum(p, acc):
                return acc + cmrg[p, sl]

            csum[sl] = lax.fori_loop(0, NS, accum, zeros16)
            return carry2

        lax.fori_loop(0, CHUNK // 16, colsum, 0)

        def row16(i, carry2):
            inv16 = 1.0 / jnp.maximum(csum[pl.ds(i * 16, 16)], 1.0)
            for r in range(16):
                inv = jnp.broadcast_to(inv16[r], (16,))
                for k in range(H // 16):
                    sl = pl.ds(k * 16, 16)
                    outv[i * 16 + r, sl] = accv[i * 16 + r, sl] * inv
            return carry2

        lax.fori_loop(0, CHUNK // 16, row16, 0)
        pltpu.sync_copy(outv, out.at[c].at[pl.ds(r0, CHUNK)])
        return carry

    lax.fori_loop(0, NCHUNK, chunk, 0)


def _edge_splits(ei, src_off):
    ei = ei.astype(jnp.int32)
    src = ei[0].reshape(NS, E // NS) + src_off
    dst = ei[1].reshape(NS, E // NS)
    pad = EP_TILE - E // NS
    src = jnp.pad(src, ((0, 0), (0, pad)),
                  constant_values=src_off)
    dst = jnp.pad(dst, ((0, 0), (0, pad)),
                  constant_values=N_NODES)
    packed = jnp.bitwise_or(jnp.left_shift(dst, 15), src)
    return packed.reshape(NS, NBLK_IDX, BLK)


def kernel(x_user, x_resource, edge_index_user_accessed_resource,
           edge_index_resource_rev_accessed_user,
           Wl_ur, Wr_ur, b_ur, Wl_ru, Wr_ru, b_ru):
    wu = jnp.concatenate([Wl_ur, Wr_ru], axis=1)
    wr = jnp.concatenate([Wl_ru, Wr_ur], axis=1)
    zeros_h = jnp.zeros((H,), jnp.float32)
    bu = jnp.concatenate([zeros_h, b_ru])[None, :]
    br = jnp.concatenate([zeros_h, b_ur])[None, :]

    y_tab, z_all = _dense_project(x_user, x_resource, wu, wr, bu, br)
    y_flat = y_tab.reshape(2 * N_NODES, H)

    pa = _edge_splits(edge_index_user_accessed_resource, 0)
    pb = _edge_splits(edge_index_resource_rev_accessed_user, N_NODES)
    p_all = jnp.stack([pa, pb])

    out_sc, _ = _sc_aggregate(y_flat, p_all)
    out_user, out_res = _split_outputs(out_sc, z_all)
    return (out_user, out_res)
